# Initial kernel scaffold; baseline (speedup 1.0000x reference)
import jax
import jax.numpy as jnp
from jax.experimental import pallas as pl


def kernel(x, edge_index, W_l, b_l, W_r, b_r, att, bias_gat, W_lin, b_lin):
    raise NotImplementedError("write your pallas kernel here")


# speedup vs baseline: 8.8604x; 8.8604x over previous
import functools

import jax
import jax.numpy as jnp
from jax import lax
from jax.experimental import pallas as pl
from jax.experimental.pallas import tpu as pltpu
from jax.experimental.pallas import tpu_sc as plsc

N = 10000
F = 128
C = 40
NP = 10240
CP = 48
K = 128
NWORK = 32
ROWS_PER_TILE = NP // 16


def _mm_body(x_ref, wl_ref, bl_ref, wr_ref, br_ref, wlin_ref, blin_ref,
             xl_ref, xr_ref, lm_ref):
    i = pl.program_id(0)
    xb = x_ref[...]
    rows = i * xb.shape[0] + lax.broadcasted_iota(jnp.int32, (xb.shape[0], 1), 0)
    mask = (rows < N).astype(jnp.float32)
    xl_ref[...] = (jnp.dot(xb, wl_ref[...],
                           preferred_element_type=jnp.float32) + bl_ref[...]) * mask
    xr_ref[...] = (jnp.dot(xb, wr_ref[...],
                           preferred_element_type=jnp.float32) + br_ref[...]) * mask
    lm_ref[...] = jnp.dot(xb, wlin_ref[...],
                          preferred_element_type=jnp.float32) + blin_ref[...]


def _combine_body(n0_ref, n1_ref, d0_ref, d1_ref, bias_ref, out_ref):
    denom = d0_ref[...] + d1_ref[...] + 1e-16
    out_ref[...] = (n0_ref[...] + n1_ref[...]) / denom + bias_ref[...]


def _sc_body(n_chunks, src_hbm, dst_hbm, xl_hbm, xr_hbm, att_hbm,
             numer_out, denom_out,
             att_v, src_v, dst_v, rows_l, rows_r, exbuf, zbuf,
             numer_sh, denom_sh, sem1, sem2):
    cid = lax.axis_index("c")
    sid = lax.axis_index("s")
    wid = sid * 2 + cid

    pltpu.sync_copy(att_hbm, att_v)

    zero16 = jnp.zeros((16,), jnp.float32)

    def zrow(j, _):
        for t in range(CP // 16):
            zbuf[j, pl.ds(t * 16, 16)] = zero16
        return 0

    lax.fori_loop(0, K, zrow, 0)
    for t in range(K // 16):
        exbuf[pl.ds(t * 16, 16)] = zero16
    for k in range(ROWS_PER_TILE // K):
        pltpu.sync_copy(zbuf, numer_sh.at[pl.ds(sid * ROWS_PER_TILE + k * K, K), :])
        pltpu.sync_copy(exbuf, denom_sh.at[pl.ds(sid * ROWS_PER_TILE + k * K, K)])
    plsc.subcore_barrier()

    iota16 = lax.iota(jnp.int32, 16)
    rowid = [iota16 + 16 * g for g in range(K // 16)]
    base = wid * (n_chunks * K)

    def chunk(j, _):
        off = base + j * K
        pltpu.sync_copy(src_hbm.at[pl.ds(off, K)], src_v)
        pltpu.sync_copy(dst_hbm.at[pl.ds(off, K)], dst_v)
        cp1 = pltpu.async_copy(xl_hbm.at[src_v], rows_l, sem1)
        cp2 = pltpu.async_copy(xr_hbm.at[dst_v], rows_r, sem2)
        cp1.wait()
        cp2.wait()

        def cbody(ci, accs):
            colv = jnp.full((16,), ci, jnp.int32)
            attv = plsc.load_gather(att_v, [colv])
            out = []
            for g in range(K // 16):
                a = plsc.load_gather(rows_l, [rowid[g], colv])
                b = plsc.load_gather(rows_r, [rowid[g], colv])
                t = a + b
                t = jnp.maximum(t, 0.2 * t)
                out.append(accs[g] + attv * t)
            return tuple(out)

        accs = lax.fori_loop(
            0, C, cbody,
            tuple(jnp.zeros((16,), jnp.float32) for _ in range(K // 16)))
        exs = [jnp.exp(a) for a in accs]
        for g in range(K // 16):
            exbuf[pl.ds(16 * g, 16)] = exs[g]

        def sbody(ci, _):
            colv = jnp.full((16,), ci, jnp.int32)
            for g in range(K // 16):
                v = plsc.load_gather(rows_l, [rowid[g], colv])
                plsc.store_scatter(rows_l, [rowid[g], colv], v * exs[g])
            return 0

        lax.fori_loop(0, C, sbody, 0)

        pltpu.sync_copy(exbuf, denom_sh.at[dst_v], add=True)
        pltpu.sync_copy(rows_l, numer_sh.at[dst_v], add=True)
        return 0

    lax.fori_loop(0, n_chunks, chunk, 0)
    plsc.subcore_barrier()

    row0 = sid * ROWS_PER_TILE
    pltpu.sync_copy(numer_sh.at[pl.ds(row0, ROWS_PER_TILE), :],
                    numer_out.at[cid, pl.ds(row0, ROWS_PER_TILE), :])
    pltpu.sync_copy(denom_sh.at[pl.ds(row0, ROWS_PER_TILE)],
                    denom_out.at[cid, pl.ds(row0, ROWS_PER_TILE)])


def kernel(x, edge_index, W_l, b_l, W_r, b_r, att, bias_gat, W_lin, b_lin):
    e_tot = edge_index.shape[1] + N
    n_chunks = -(-e_tot // (NWORK * K))
    e_pad = NWORK * K * n_chunks

    src = edge_index[0]
    dst = edge_index[1]
    loop = jnp.arange(N, dtype=jnp.int32)
    fill = jnp.full((e_pad - e_tot,), NP - 1, jnp.int32)
    src_pad = jnp.concatenate([src, loop, fill])
    dst_pad = jnp.concatenate([dst, loop, fill])
    x_pad = jnp.pad(x, ((0, NP - N), (0, 0)))
    wl_p = jnp.pad(W_l, ((0, 0), (0, CP - C)))
    wr_p = jnp.pad(W_r, ((0, 0), (0, CP - C)))
    wlin_p = jnp.pad(W_lin, ((0, 0), (0, CP - C)))
    bl_p = jnp.pad(b_l, (0, CP - C)).reshape(1, CP)
    br_p = jnp.pad(b_r, (0, CP - C)).reshape(1, CP)
    blin_p = jnp.pad(b_lin, (0, CP - C)).reshape(1, CP)
    att_p = jnp.pad(att, (0, 128 - C))
    bias_p = jnp.pad(bias_gat, (0, CP - C)).reshape(1, CP)

    blk = ROWS_PER_TILE
    grid = NP // blk
    xl_pad, xr_pad, lm_pad = pl.pallas_call(
        _mm_body,
        grid=(grid,),
        in_specs=[
            pl.BlockSpec((blk, F), lambda i: (i, 0)),
            pl.BlockSpec((F, CP), lambda i: (0, 0)),
            pl.BlockSpec((1, CP), lambda i: (0, 0)),
            pl.BlockSpec((F, CP), lambda i: (0, 0)),
            pl.BlockSpec((1, CP), lambda i: (0, 0)),
            pl.BlockSpec((F, CP), lambda i: (0, 0)),
            pl.BlockSpec((1, CP), lambda i: (0, 0)),
        ],
        out_specs=[
            pl.BlockSpec((blk, CP), lambda i: (i, 0)),
            pl.BlockSpec((blk, CP), lambda i: (i, 0)),
            pl.BlockSpec((blk, CP), lambda i: (i, 0)),
        ],
        out_shape=[
            jax.ShapeDtypeStruct((NP, CP), jnp.float32),
            jax.ShapeDtypeStruct((NP, CP), jnp.float32),
            jax.ShapeDtypeStruct((NP, CP), jnp.float32),
        ],
    )(x_pad, wl_p, bl_p, wr_p, br_p, wlin_p, blin_p)

    mesh = plsc.VectorSubcoreMesh(core_axis_name="c", subcore_axis_name="s")
    numer_part, denom_part = pl.kernel(
        functools.partial(_sc_body, n_chunks),
        out_type=(
            jax.ShapeDtypeStruct((2, NP, CP), jnp.float32),
            jax.ShapeDtypeStruct((2, NP), jnp.float32),
        ),
        mesh=mesh,
        scratch_types=[
            pltpu.VMEM((128,), jnp.float32),
            pltpu.VMEM((K,), jnp.int32),
            pltpu.VMEM((K,), jnp.int32),
            pltpu.VMEM((K, CP), jnp.float32),
            pltpu.VMEM((K, CP), jnp.float32),
            pltpu.VMEM((K,), jnp.float32),
            pltpu.VMEM((K, CP), jnp.float32),
            pltpu.VMEM_SHARED((NP, CP), jnp.float32),
            pltpu.VMEM_SHARED((NP,), jnp.float32),
            pltpu.SemaphoreType.DMA,
            pltpu.SemaphoreType.DMA,
        ],
        compiler_params=pltpu.CompilerParams(needs_layout_passes=False,
                                             use_tc_tiling_on_sc=False),
    )(src_pad, dst_pad, xl_pad, xr_pad, att_p)

    out_comb = pl.pallas_call(
        _combine_body,
        grid=(grid,),
        in_specs=[
            pl.BlockSpec((blk, CP), lambda i: (i, 0)),
            pl.BlockSpec((blk, CP), lambda i: (i, 0)),
            pl.BlockSpec((blk, 1), lambda i: (i, 0)),
            pl.BlockSpec((blk, 1), lambda i: (i, 0)),
            pl.BlockSpec((1, CP), lambda i: (0, 0)),
        ],
        out_specs=pl.BlockSpec((blk, CP), lambda i: (i, 0)),
        out_shape=jax.ShapeDtypeStruct((NP, CP), jnp.float32),
    )(numer_part[0], numer_part[1],
      denom_part[0].reshape(NP, 1), denom_part[1].reshape(NP, 1), bias_p)

    out_gnn = out_comb[:N, :C]
    out_lm = lm_pad[:N, :C]
    return (out_gnn, out_lm, x)


# speedup vs baseline: 11.2116x; 1.2654x over previous
import functools

import jax
import jax.numpy as jnp
from jax import lax
from jax.experimental import pallas as pl
from jax.experimental.pallas import tpu as pltpu
from jax.experimental.pallas import tpu_sc as plsc

N = 10000
F = 128
C = 40
NP = 10240
CP = 48
K = 128
NWORK = 32
ROWS_PER_TILE = NP // 16


def _mm_body(x_ref, wl_ref, bl_ref, wr_ref, br_ref, wlin_ref, blin_ref,
             xl_ref, xr_ref, lm_ref):
    i = pl.program_id(0)
    xb = x_ref[...]
    rows = i * xb.shape[0] + lax.broadcasted_iota(jnp.int32, (xb.shape[0], 1), 0)
    mask = (rows < N).astype(jnp.float32)
    xl_ref[...] = (jnp.dot(xb, wl_ref[...],
                           preferred_element_type=jnp.float32) + bl_ref[...]) * mask
    xr_ref[...] = (jnp.dot(xb, wr_ref[...],
                           preferred_element_type=jnp.float32) + br_ref[...]) * mask
    lm_ref[...] = jnp.dot(xb, wlin_ref[...],
                          preferred_element_type=jnp.float32) + blin_ref[...]


def _combine_body(n0_ref, n1_ref, d0_ref, d1_ref, bias_ref, out_ref):
    denom = d0_ref[...] + d1_ref[...] + 1e-16
    out_ref[...] = (n0_ref[...] + n1_ref[...]) / denom + bias_ref[...]


def _sc_body(n_chunks, src_hbm, dst_hbm, xl_hbm, xr_hbm, att_hbm,
             numer_out, denom_out,
             att_v, src_v, dst_v, rows_l0, rows_l1, rows_r0, rows_r1,
             exbuf, zbuf, numer_sh, denom_sh, semg0, semg1):
    cid = lax.axis_index("c")
    sid = lax.axis_index("s")
    wid = sid * 2 + cid
    rows_l = (rows_l0, rows_l1)
    rows_r = (rows_r0, rows_r1)
    semg = (semg0, semg1)

    pltpu.sync_copy(att_hbm, att_v)
    pltpu.sync_copy(src_hbm.at[wid], src_v)
    pltpu.sync_copy(dst_hbm.at[wid], dst_v)

    zero16 = jnp.zeros((16,), jnp.float32)

    def zrow(j, _):
        for t in range(CP // 16):
            zbuf[j, pl.ds(t * 16, 16)] = zero16
        return 0

    lax.fori_loop(0, K, zrow, 0)
    for t in range(K // 16):
        exbuf[pl.ds(t * 16, 16)] = zero16
    for k in range(ROWS_PER_TILE // K):
        pltpu.sync_copy(zbuf, numer_sh.at[pl.ds(sid * ROWS_PER_TILE + k * K, K), :])
        pltpu.sync_copy(exbuf, denom_sh.at[pl.ds(sid * ROWS_PER_TILE + k * K, K)])
    plsc.subcore_barrier()

    iota16 = lax.iota(jnp.int32, 16)
    rowid = [iota16 + 16 * g for g in range(K // 16)]

    def issue_gather(j, b):
        pltpu.async_copy(xl_hbm.at[src_v.at[j]], rows_l[b], semg[b])
        pltpu.async_copy(xr_hbm.at[dst_v.at[j]], rows_r[b], semg[b])

    def wait_gather(b):
        pltpu.make_async_copy(xl_hbm.at[src_v.at[0]], rows_l[b], semg[b]).wait()
        pltpu.make_async_copy(xr_hbm.at[dst_v.at[0]], rows_r[b], semg[b]).wait()

    def do_chunk(j, b):
        wait_gather(b)

        @pl.when(j + 1 < n_chunks)
        def _():
            issue_gather(j + 1, b ^ 1)

        def cbody(ci, accs):
            colv = jnp.full((16,), ci, jnp.int32)
            attv = plsc.load_gather(att_v, [colv])
            out = []
            for g in range(K // 16):
                a = plsc.load_gather(rows_l[b], [rowid[g], colv])
                r = plsc.load_gather(rows_r[b], [rowid[g], colv])
                t = a + r
                t = jnp.maximum(t, 0.2 * t)
                out.append(accs[g] + attv * t)
            return tuple(out)

        accs = lax.fori_loop(
            0, C, cbody,
            tuple(jnp.zeros((16,), jnp.float32) for _ in range(K // 16)))
        exs = [jnp.exp(a) for a in accs]
        for g in range(K // 16):
            exbuf[pl.ds(16 * g, 16)] = exs[g]

        def sbody(ci, _):
            colv = jnp.full((16,), ci, jnp.int32)
            for g in range(K // 16):
                v = plsc.load_gather(rows_l[b], [rowid[g], colv])
                plsc.store_scatter(rows_l[b], [rowid[g], colv], v * exs[g])
            return 0

        lax.fori_loop(0, C, sbody, 0)

        pltpu.sync_copy(exbuf, denom_sh.at[dst_v.at[j]], add=True)
        pltpu.sync_copy(rows_l[b], numer_sh.at[dst_v.at[j]], add=True)

    issue_gather(0, 0)

    def outer(i, _):
        for b in range(2):
            do_chunk(2 * i + b, b)
        return 0

    lax.fori_loop(0, n_chunks // 2, outer, 0)
    plsc.subcore_barrier()

    row0 = sid * ROWS_PER_TILE
    pltpu.sync_copy(numer_sh.at[pl.ds(row0, ROWS_PER_TILE), :],
                    numer_out.at[cid, pl.ds(row0, ROWS_PER_TILE), :])
    pltpu.sync_copy(denom_sh.at[pl.ds(row0, ROWS_PER_TILE)],
                    denom_out.at[cid, pl.ds(row0, ROWS_PER_TILE)])


def kernel(x, edge_index, W_l, b_l, W_r, b_r, att, bias_gat, W_lin, b_lin):
    e_tot = edge_index.shape[1] + N
    n_chunks = -(-e_tot // (NWORK * K))
    n_chunks += n_chunks % 2
    e_pad = NWORK * K * n_chunks

    src = edge_index[0]
    dst = edge_index[1]
    loop = jnp.arange(N, dtype=jnp.int32)
    fill = jnp.full((e_pad - e_tot,), NP - 1, jnp.int32)
    src_pad = jnp.concatenate([src, loop, fill]).reshape(NWORK, n_chunks, K)
    dst_pad = jnp.concatenate([dst, loop, fill]).reshape(NWORK, n_chunks, K)
    x_pad = jnp.pad(x, ((0, NP - N), (0, 0)))
    wl_p = jnp.pad(W_l, ((0, 0), (0, CP - C)))
    wr_p = jnp.pad(W_r, ((0, 0), (0, CP - C)))
    wlin_p = jnp.pad(W_lin, ((0, 0), (0, CP - C)))
    bl_p = jnp.pad(b_l, (0, CP - C)).reshape(1, CP)
    br_p = jnp.pad(b_r, (0, CP - C)).reshape(1, CP)
    blin_p = jnp.pad(b_lin, (0, CP - C)).reshape(1, CP)
    att_p = jnp.pad(att, (0, 128 - C))
    bias_p = jnp.pad(bias_gat, (0, CP - C)).reshape(1, CP)

    blk = ROWS_PER_TILE
    grid = NP // blk
    xl_pad, xr_pad, lm_pad = pl.pallas_call(
        _mm_body,
        grid=(grid,),
        in_specs=[
            pl.BlockSpec((blk, F), lambda i: (i, 0)),
            pl.BlockSpec((F, CP), lambda i: (0, 0)),
            pl.BlockSpec((1, CP), lambda i: (0, 0)),
            pl.BlockSpec((F, CP), lambda i: (0, 0)),
            pl.BlockSpec((1, CP), lambda i: (0, 0)),
            pl.BlockSpec((F, CP), lambda i: (0, 0)),
            pl.BlockSpec((1, CP), lambda i: (0, 0)),
        ],
        out_specs=[
            pl.BlockSpec((blk, CP), lambda i: (i, 0)),
            pl.BlockSpec((blk, CP), lambda i: (i, 0)),
            pl.BlockSpec((blk, CP), lambda i: (i, 0)),
        ],
        out_shape=[
            jax.ShapeDtypeStruct((NP, CP), jnp.float32),
            jax.ShapeDtypeStruct((NP, CP), jnp.float32),
            jax.ShapeDtypeStruct((NP, CP), jnp.float32),
        ],
    )(x_pad, wl_p, bl_p, wr_p, br_p, wlin_p, blin_p)

    mesh = plsc.VectorSubcoreMesh(core_axis_name="c", subcore_axis_name="s")
    numer_part, denom_part = pl.kernel(
        functools.partial(_sc_body, n_chunks),
        out_type=(
            jax.ShapeDtypeStruct((2, NP, CP), jnp.float32),
            jax.ShapeDtypeStruct((2, NP), jnp.float32),
        ),
        mesh=mesh,
        scratch_types=[
            pltpu.VMEM((128,), jnp.float32),
            pltpu.VMEM((n_chunks, K), jnp.int32),
            pltpu.VMEM((n_chunks, K), jnp.int32),
            pltpu.VMEM((K, CP), jnp.float32),
            pltpu.VMEM((K, CP), jnp.float32),
            pltpu.VMEM((K, CP), jnp.float32),
            pltpu.VMEM((K, CP), jnp.float32),
            pltpu.VMEM((K,), jnp.float32),
            pltpu.VMEM((K, CP), jnp.float32),
            pltpu.VMEM_SHARED((NP, CP), jnp.float32),
            pltpu.VMEM_SHARED((NP,), jnp.float32),
            pltpu.SemaphoreType.DMA,
            pltpu.SemaphoreType.DMA,
        ],
        compiler_params=pltpu.CompilerParams(needs_layout_passes=False,
                                             use_tc_tiling_on_sc=False),
    )(src_pad, dst_pad, xl_pad, xr_pad, att_p)

    out_comb = pl.pallas_call(
        _combine_body,
        grid=(grid,),
        in_specs=[
            pl.BlockSpec((blk, CP), lambda i: (i, 0)),
            pl.BlockSpec((blk, CP), lambda i: (i, 0)),
            pl.BlockSpec((blk, 1), lambda i: (i, 0)),
            pl.BlockSpec((blk, 1), lambda i: (i, 0)),
            pl.BlockSpec((1, CP), lambda i: (0, 0)),
        ],
        out_specs=pl.BlockSpec((blk, CP), lambda i: (i, 0)),
        out_shape=jax.ShapeDtypeStruct((NP, CP), jnp.float32),
    )(numer_part[0], numer_part[1],
      denom_part[0].reshape(NP, 1), denom_part[1].reshape(NP, 1), bias_p)

    out_gnn = out_comb[:N, :C]
    out_lm = lm_pad[:N, :C]
    return (out_gnn, out_lm, x)


# speedup vs baseline: 11.3223x; 1.0099x over previous
import functools

import jax
import jax.numpy as jnp
from jax import lax
from jax.experimental import pallas as pl
from jax.experimental.pallas import tpu as pltpu
from jax.experimental.pallas import tpu_sc as plsc

N = 10000
F = 128
C = 40
NP = 10240
CP = 48
K = 128
NWORK = 32
ROWS_PER_TILE = NP // 16


def _mm_body(x_ref, wl_ref, bl_ref, wr_ref, br_ref, wlin_ref, blin_ref,
             xl_ref, xr_ref, lm_ref):
    i = pl.program_id(0)
    xb = x_ref[...]
    rows = i * xb.shape[0] + lax.broadcasted_iota(jnp.int32, (xb.shape[0], 1), 0)
    mask = (rows < N).astype(jnp.float32)
    xl_ref[...] = (jnp.dot(xb, wl_ref[...],
                           preferred_element_type=jnp.float32) + bl_ref[...]) * mask
    xr_ref[...] = (jnp.dot(xb, wr_ref[...],
                           preferred_element_type=jnp.float32) + br_ref[...]) * mask
    lm_ref[...] = jnp.dot(xb, wlin_ref[...],
                          preferred_element_type=jnp.float32) + blin_ref[...]


def _combine_body(n0_ref, n1_ref, d0_ref, d1_ref, bias_ref, out_ref):
    denom = d0_ref[...] + d1_ref[...] + 1e-16
    out_ref[...] = (n0_ref[...] + n1_ref[...]) / denom + bias_ref[...]


def _sc_body(n_chunks, src_hbm, dst_hbm, xl_hbm, xr_hbm, att_hbm,
             numer_out,
             att_v, src_v, dst_v, rows_l0, rows_l1, rows_r0, rows_r1,
             zbuf, numer_sh, semg0, semg1):
    cid = lax.axis_index("c")
    sid = lax.axis_index("s")
    wid = sid * 2 + cid
    rows_l = (rows_l0, rows_l1)
    rows_r = (rows_r0, rows_r1)
    semg = (semg0, semg1)

    pltpu.sync_copy(att_hbm, att_v)
    pltpu.sync_copy(src_hbm.at[wid], src_v)
    pltpu.sync_copy(dst_hbm.at[wid], dst_v)

    zero16 = jnp.zeros((16,), jnp.float32)

    def zrow(j, _):
        for t in range(CP // 16):
            zbuf[j, pl.ds(t * 16, 16)] = zero16
        return 0

    lax.fori_loop(0, K, zrow, 0)
    for k in range(ROWS_PER_TILE // K):
        pltpu.sync_copy(zbuf, numer_sh.at[pl.ds(sid * ROWS_PER_TILE + k * K, K), :])
    plsc.subcore_barrier()

    iota16 = lax.iota(jnp.int32, 16)
    rowid = [iota16 + 16 * g for g in range(K // 16)]

    def issue_gather(j, b):
        pltpu.async_copy(xl_hbm.at[src_v.at[j]], rows_l[b], semg[b])
        pltpu.async_copy(xr_hbm.at[dst_v.at[j]], rows_r[b], semg[b])

    def wait_gather(b):
        pltpu.make_async_copy(xl_hbm.at[src_v.at[0]], rows_l[b], semg[b]).wait()
        pltpu.make_async_copy(xr_hbm.at[dst_v.at[0]], rows_r[b], semg[b]).wait()

    def do_chunk(j, b):
        wait_gather(b)

        @pl.when(j + 1 < n_chunks)
        def _():
            issue_gather(j + 1, b ^ 1)

        def cbody(ci, accs):
            colv = jnp.full((16,), ci, jnp.int32)
            attv = plsc.load_gather(att_v, [colv])
            out = []
            for g in range(K // 16):
                a = plsc.load_gather(rows_l[b], [rowid[g], colv])
                r = plsc.load_gather(rows_r[b], [rowid[g], colv])
                t = a + r
                t = jnp.maximum(t, 0.2 * t)
                out.append(accs[g] + attv * t)
            return tuple(out)

        accs = lax.fori_loop(
            0, C, cbody,
            tuple(jnp.zeros((16,), jnp.float32) for _ in range(K // 16)))
        exs = [jnp.exp(a) for a in accs]
        col40 = jnp.full((16,), C, jnp.int32)
        for g in range(K // 16):
            plsc.store_scatter(rows_l[b], [rowid[g], col40], exs[g])

        def sbody(ci, _):
            colv = jnp.full((16,), ci, jnp.int32)
            for g in range(K // 16):
                v = plsc.load_gather(rows_l[b], [rowid[g], colv])
                plsc.store_scatter(rows_l[b], [rowid[g], colv], v * exs[g])
            return 0

        lax.fori_loop(0, C, sbody, 0)

        pltpu.sync_copy(rows_l[b], numer_sh.at[dst_v.at[j]], add=True)

    issue_gather(0, 0)

    def outer(i, _):
        for b in range(2):
            do_chunk(2 * i + b, b)
        return 0

    lax.fori_loop(0, n_chunks // 2, outer, 0)
    plsc.subcore_barrier()

    row0 = sid * ROWS_PER_TILE
    pltpu.sync_copy(numer_sh.at[pl.ds(row0, ROWS_PER_TILE), :],
                    numer_out.at[cid, pl.ds(row0, ROWS_PER_TILE), :])


def kernel(x, edge_index, W_l, b_l, W_r, b_r, att, bias_gat, W_lin, b_lin):
    e_tot = edge_index.shape[1] + N
    n_chunks = -(-e_tot // (NWORK * K))
    n_chunks += n_chunks % 2
    e_pad = NWORK * K * n_chunks

    src = edge_index[0]
    dst = edge_index[1]
    loop = jnp.arange(N, dtype=jnp.int32)
    fill = jnp.full((e_pad - e_tot,), NP - 1, jnp.int32)
    src_pad = jnp.concatenate([src, loop, fill]).reshape(NWORK, n_chunks, K)
    dst_pad = jnp.concatenate([dst, loop, fill]).reshape(NWORK, n_chunks, K)
    x_pad = jnp.pad(x, ((0, NP - N), (0, 0)))
    wl_p = jnp.pad(W_l, ((0, 0), (0, CP - C)))
    wr_p = jnp.pad(W_r, ((0, 0), (0, CP - C)))
    wlin_p = jnp.pad(W_lin, ((0, 0), (0, CP - C)))
    bl_p = jnp.pad(b_l, (0, CP - C)).reshape(1, CP)
    br_p = jnp.pad(b_r, (0, CP - C)).reshape(1, CP)
    blin_p = jnp.pad(b_lin, (0, CP - C)).reshape(1, CP)
    att_p = jnp.pad(att, (0, 128 - C))
    bias_p = jnp.pad(bias_gat, (0, CP - C)).reshape(1, CP)

    blk = ROWS_PER_TILE
    grid = NP // blk
    xl_pad, xr_pad, lm_pad = pl.pallas_call(
        _mm_body,
        grid=(grid,),
        in_specs=[
            pl.BlockSpec((blk, F), lambda i: (i, 0)),
            pl.BlockSpec((F, CP), lambda i: (0, 0)),
            pl.BlockSpec((1, CP), lambda i: (0, 0)),
            pl.BlockSpec((F, CP), lambda i: (0, 0)),
            pl.BlockSpec((1, CP), lambda i: (0, 0)),
            pl.BlockSpec((F, CP), lambda i: (0, 0)),
            pl.BlockSpec((1, CP), lambda i: (0, 0)),
        ],
        out_specs=[
            pl.BlockSpec((blk, CP), lambda i: (i, 0)),
            pl.BlockSpec((blk, CP), lambda i: (i, 0)),
            pl.BlockSpec((blk, CP), lambda i: (i, 0)),
        ],
        out_shape=[
            jax.ShapeDtypeStruct((NP, CP), jnp.float32),
            jax.ShapeDtypeStruct((NP, CP), jnp.float32),
            jax.ShapeDtypeStruct((NP, CP), jnp.float32),
        ],
    )(x_pad, wl_p, bl_p, wr_p, br_p, wlin_p, blin_p)

    mesh = plsc.VectorSubcoreMesh(core_axis_name="c", subcore_axis_name="s")
    numer_part = pl.kernel(
        functools.partial(_sc_body, n_chunks),
        out_type=jax.ShapeDtypeStruct((2, NP, CP), jnp.float32),
        mesh=mesh,
        scratch_types=[
            pltpu.VMEM((128,), jnp.float32),
            pltpu.VMEM((n_chunks, K), jnp.int32),
            pltpu.VMEM((n_chunks, K), jnp.int32),
            pltpu.VMEM((K, CP), jnp.float32),
            pltpu.VMEM((K, CP), jnp.float32),
            pltpu.VMEM((K, CP), jnp.float32),
            pltpu.VMEM((K, CP), jnp.float32),
            pltpu.VMEM((K, CP), jnp.float32),
            pltpu.VMEM_SHARED((NP, CP), jnp.float32),
            pltpu.SemaphoreType.DMA,
            pltpu.SemaphoreType.DMA,
        ],
        compiler_params=pltpu.CompilerParams(needs_layout_passes=False,
                                             use_tc_tiling_on_sc=False),
    )(src_pad, dst_pad, xl_pad, xr_pad, att_p)

    out_comb = pl.pallas_call(
        _combine_body,
        grid=(grid,),
        in_specs=[
            pl.BlockSpec((blk, CP), lambda i: (i, 0)),
            pl.BlockSpec((blk, CP), lambda i: (i, 0)),
            pl.BlockSpec((blk, 1), lambda i: (i, 0)),
            pl.BlockSpec((blk, 1), lambda i: (i, 0)),
            pl.BlockSpec((1, CP), lambda i: (0, 0)),
        ],
        out_specs=pl.BlockSpec((blk, CP), lambda i: (i, 0)),
        out_shape=jax.ShapeDtypeStruct((NP, CP), jnp.float32),
    )(numer_part[0], numer_part[1],
      numer_part[0, :, C].reshape(NP, 1), numer_part[1, :, C].reshape(NP, 1),
      bias_p)

    out_gnn = out_comb[:N, :C]
    out_lm = lm_pad[:N, :C]
    return (out_gnn, out_lm, x)


# speedup vs baseline: 15.3251x; 1.3535x over previous
import functools

import jax
import jax.numpy as jnp
from jax import lax
from jax.experimental import pallas as pl
from jax.experimental.pallas import tpu as pltpu
from jax.experimental.pallas import tpu_sc as plsc

N = 10000
F = 128
C = 40
NP = 10240
CP = 48
K = 128
NWORK = 32
ROWS_PER_TILE = NP // 16


def _mm_body(x_ref, wl_ref, bl_ref, wr_ref, br_ref, wlin_ref, blin_ref,
             xl_ref, xr_ref, lm_ref):
    i = pl.program_id(0)
    xb = x_ref[...]
    rows = i * xb.shape[0] + lax.broadcasted_iota(jnp.int32, (xb.shape[0], 1), 0)
    mask = (rows < N).astype(jnp.float32)
    xl_ref[...] = (jnp.dot(xb, wl_ref[...],
                           preferred_element_type=jnp.float32) + bl_ref[...]) * mask
    xr_ref[...] = (jnp.dot(xb, wr_ref[...],
                           preferred_element_type=jnp.float32) + br_ref[...]) * mask
    lm_ref[...] = jnp.dot(xb, wlin_ref[...],
                          preferred_element_type=jnp.float32) + blin_ref[...]


def _combine_body(n0_ref, n1_ref, d0_ref, d1_ref, bias_ref, out_ref):
    denom = d0_ref[...] + d1_ref[...] + 1e-16
    out_ref[...] = (n0_ref[...] + n1_ref[...]) / denom + bias_ref[...]


def _sc_body(n_chunks, src_hbm, dst_hbm, xl_hbm, xr_hbm, att_hbm,
             numer_out,
             att_v, src_v, dst_v, rows_l0, rows_l1, rows_r0, rows_r1,
             zbuf, numer_sh, semg0, semg1):
    cid = lax.axis_index("c")
    sid = lax.axis_index("s")
    wid = sid * 2 + cid
    rows_l = (rows_l0, rows_l1)
    rows_r = (rows_r0, rows_r1)
    semg = (semg0, semg1)

    pltpu.sync_copy(att_hbm, att_v)
    pltpu.sync_copy(src_hbm.at[wid], src_v)
    pltpu.sync_copy(dst_hbm.at[wid], dst_v)

    zero16 = jnp.zeros((16,), jnp.float32)

    def zrow(j, _):
        for t in range(CP // 16):
            zbuf[j, pl.ds(t * 16, 16)] = zero16
        return 0

    lax.fori_loop(0, K, zrow, 0)
    for k in range(ROWS_PER_TILE // K):
        pltpu.sync_copy(zbuf, numer_sh.at[pl.ds(sid * ROWS_PER_TILE + k * K, K), :])
    plsc.subcore_barrier()

    iota16 = lax.iota(jnp.int32, 16)
    rowid = [iota16 + 16 * g for g in range(K // 16)]

    def issue_gather(j, b):
        pltpu.async_copy(xl_hbm.at[src_v.at[j]], rows_l[b], semg[b])
        pltpu.async_copy(xr_hbm.at[dst_v.at[j]], rows_r[b], semg[b])

    def wait_gather(b):
        pltpu.make_async_copy(xl_hbm.at[src_v.at[0]], rows_l[b], semg[b]).wait()
        pltpu.make_async_copy(xr_hbm.at[dst_v.at[0]], rows_r[b], semg[b]).wait()

    def do_chunk(j, b):
        wait_gather(b)

        @pl.when(j + 1 < n_chunks)
        def _():
            issue_gather(j + 1, b ^ 1)

        accs0 = tuple(jnp.zeros((16,), jnp.float32) for _ in range(K // 16))

        @plsc.parallel_loop(0, C, unroll=4, carry=accs0)
        def accs(ci, acc_in):
            colv = jnp.full((16,), ci, jnp.int32)
            attv = plsc.load_gather(att_v, [colv])
            out = []
            for g in range(K // 16):
                a = plsc.load_gather(rows_l[b], [rowid[g], colv])
                r = plsc.load_gather(rows_r[b], [rowid[g], colv])
                t = a + r
                t = jnp.maximum(t, 0.2 * t)
                out.append(acc_in[g] + attv * t)
            return tuple(out)

        exs = [jnp.exp(a) for a in accs]
        col40 = jnp.full((16,), C, jnp.int32)
        for g in range(K // 16):
            plsc.store_scatter(rows_l[b], [rowid[g], col40], exs[g])

        @plsc.parallel_loop(0, C, unroll=4)
        def _scale(ci):
            colv = jnp.full((16,), ci, jnp.int32)
            for g in range(K // 16):
                v = plsc.load_gather(rows_l[b], [rowid[g], colv])
                plsc.store_scatter(rows_l[b], [rowid[g], colv], v * exs[g])

        pltpu.sync_copy(rows_l[b], numer_sh.at[dst_v.at[j]], add=True)

    issue_gather(0, 0)

    def outer(i, _):
        for b in range(2):
            do_chunk(2 * i + b, b)
        return 0

    lax.fori_loop(0, n_chunks // 2, outer, 0)
    plsc.subcore_barrier()

    row0 = sid * ROWS_PER_TILE
    pltpu.sync_copy(numer_sh.at[pl.ds(row0, ROWS_PER_TILE), :],
                    numer_out.at[cid, pl.ds(row0, ROWS_PER_TILE), :])


def kernel(x, edge_index, W_l, b_l, W_r, b_r, att, bias_gat, W_lin, b_lin):
    e_tot = edge_index.shape[1] + N
    n_chunks = -(-e_tot // (NWORK * K))
    n_chunks += n_chunks % 2
    e_pad = NWORK * K * n_chunks

    src = edge_index[0]
    dst = edge_index[1]
    loop = jnp.arange(N, dtype=jnp.int32)
    fill = jnp.full((e_pad - e_tot,), NP - 1, jnp.int32)
    src_pad = jnp.concatenate([src, loop, fill]).reshape(NWORK, n_chunks, K)
    dst_pad = jnp.concatenate([dst, loop, fill]).reshape(NWORK, n_chunks, K)
    x_pad = jnp.pad(x, ((0, NP - N), (0, 0)))
    wl_p = jnp.pad(W_l, ((0, 0), (0, CP - C)))
    wr_p = jnp.pad(W_r, ((0, 0), (0, CP - C)))
    wlin_p = jnp.pad(W_lin, ((0, 0), (0, CP - C)))
    bl_p = jnp.pad(b_l, (0, CP - C)).reshape(1, CP)
    br_p = jnp.pad(b_r, (0, CP - C)).reshape(1, CP)
    blin_p = jnp.pad(b_lin, (0, CP - C)).reshape(1, CP)
    att_p = jnp.pad(att, (0, 128 - C))
    bias_p = jnp.pad(bias_gat, (0, CP - C)).reshape(1, CP)

    blk = ROWS_PER_TILE
    grid = NP // blk
    xl_pad, xr_pad, lm_pad = pl.pallas_call(
        _mm_body,
        grid=(grid,),
        in_specs=[
            pl.BlockSpec((blk, F), lambda i: (i, 0)),
            pl.BlockSpec((F, CP), lambda i: (0, 0)),
            pl.BlockSpec((1, CP), lambda i: (0, 0)),
            pl.BlockSpec((F, CP), lambda i: (0, 0)),
            pl.BlockSpec((1, CP), lambda i: (0, 0)),
            pl.BlockSpec((F, CP), lambda i: (0, 0)),
            pl.BlockSpec((1, CP), lambda i: (0, 0)),
        ],
        out_specs=[
            pl.BlockSpec((blk, CP), lambda i: (i, 0)),
            pl.BlockSpec((blk, CP), lambda i: (i, 0)),
            pl.BlockSpec((blk, CP), lambda i: (i, 0)),
        ],
        out_shape=[
            jax.ShapeDtypeStruct((NP, CP), jnp.float32),
            jax.ShapeDtypeStruct((NP, CP), jnp.float32),
            jax.ShapeDtypeStruct((NP, CP), jnp.float32),
        ],
    )(x_pad, wl_p, bl_p, wr_p, br_p, wlin_p, blin_p)

    mesh = plsc.VectorSubcoreMesh(core_axis_name="c", subcore_axis_name="s")
    numer_part = pl.kernel(
        functools.partial(_sc_body, n_chunks),
        out_type=jax.ShapeDtypeStruct((2, NP, CP), jnp.float32),
        mesh=mesh,
        scratch_types=[
            pltpu.VMEM((128,), jnp.float32),
            pltpu.VMEM((n_chunks, K), jnp.int32),
            pltpu.VMEM((n_chunks, K), jnp.int32),
            pltpu.VMEM((K, CP), jnp.float32),
            pltpu.VMEM((K, CP), jnp.float32),
            pltpu.VMEM((K, CP), jnp.float32),
            pltpu.VMEM((K, CP), jnp.float32),
            pltpu.VMEM((K, CP), jnp.float32),
            pltpu.VMEM_SHARED((NP, CP), jnp.float32),
            pltpu.SemaphoreType.DMA,
            pltpu.SemaphoreType.DMA,
        ],
        compiler_params=pltpu.CompilerParams(needs_layout_passes=False,
                                             use_tc_tiling_on_sc=False),
    )(src_pad, dst_pad, xl_pad, xr_pad, att_p)

    out_comb = pl.pallas_call(
        _combine_body,
        grid=(grid,),
        in_specs=[
            pl.BlockSpec((blk, CP), lambda i: (i, 0)),
            pl.BlockSpec((blk, CP), lambda i: (i, 0)),
            pl.BlockSpec((blk, 1), lambda i: (i, 0)),
            pl.BlockSpec((blk, 1), lambda i: (i, 0)),
            pl.BlockSpec((1, CP), lambda i: (0, 0)),
        ],
        out_specs=pl.BlockSpec((blk, CP), lambda i: (i, 0)),
        out_shape=jax.ShapeDtypeStruct((NP, CP), jnp.float32),
    )(numer_part[0], numer_part[1],
      numer_part[0, :, C].reshape(NP, 1), numer_part[1, :, C].reshape(NP, 1),
      bias_p)

    out_gnn = out_comb[:N, :C]
    out_lm = lm_pad[:N, :C]
    return (out_gnn, out_lm, x)


# speedup vs baseline: 20.3806x; 1.3299x over previous
import functools

import jax
import jax.numpy as jnp
from jax import lax
from jax.experimental import pallas as pl
from jax.experimental.pallas import tpu as pltpu
from jax.experimental.pallas import tpu_sc as plsc

N = 10000
F = 128
C = 40
NP = 10240
CP = 48
K = 128
NWORK = 32
ROWS_PER_TILE = NP // 16


def _mm_body(x_ref, wl_ref, bl_ref, wr_ref, br_ref, wlin_ref, blin_ref,
             xl_ref, xr_ref, lm_ref):
    i = pl.program_id(0)
    xb = x_ref[...]
    rows = i * xb.shape[0] + lax.broadcasted_iota(jnp.int32, (xb.shape[0], 1), 0)
    mask = (rows < N).astype(jnp.float32)
    xl_ref[...] = (jnp.dot(xb, wl_ref[...],
                           preferred_element_type=jnp.float32) + bl_ref[...]) * mask
    xr_ref[...] = (jnp.dot(xb, wr_ref[...],
                           preferred_element_type=jnp.float32) + br_ref[...]) * mask
    lm_ref[...] = jnp.dot(xb, wlin_ref[...],
                          preferred_element_type=jnp.float32) + blin_ref[...]


def _combine_body(n0_ref, n1_ref, d0_ref, d1_ref, bias_ref, out_ref):
    denom = d0_ref[...] + d1_ref[...] + 1e-16
    out_ref[...] = (n0_ref[...] + n1_ref[...]) / denom + bias_ref[...]


def _sc_body(n_chunks, src_hbm, dst_hbm, xl_hbm, xr_hbm, att_hbm,
             numer_out,
             att_v, src_v, dst_v, rows_l0, rows_l1, rows_r0, rows_r1,
             zbuf, numer_sh, semg0, semg1):
    cid = lax.axis_index("c")
    sid = lax.axis_index("s")
    wid = sid * 2 + cid
    rows_l = (rows_l0, rows_l1)
    rows_r = (rows_r0, rows_r1)
    semg = (semg0, semg1)

    pltpu.sync_copy(att_hbm, att_v)
    pltpu.sync_copy(src_hbm.at[wid], src_v)
    pltpu.sync_copy(dst_hbm.at[wid], dst_v)

    zero16 = jnp.zeros((16,), jnp.float32)

    def zrow(j, _):
        for t in range(CP // 16):
            zbuf[j, pl.ds(t * 16, 16)] = zero16
        return 0

    lax.fori_loop(0, K, zrow, 0)
    for k in range(ROWS_PER_TILE // K):
        pltpu.sync_copy(zbuf, numer_sh.at[pl.ds(sid * ROWS_PER_TILE + k * K, K), :])
    plsc.subcore_barrier()

    iota16 = lax.iota(jnp.int32, 16)
    rowid = [iota16 + 16 * g for g in range(K // 16)]

    def issue_gather(j, b):
        pltpu.async_copy(xl_hbm.at[src_v.at[j]], rows_l[b], semg[b])
        pltpu.async_copy(xr_hbm.at[dst_v.at[j]], rows_r[b], semg[b])

    def wait_gather(b):
        pltpu.make_async_copy(xl_hbm.at[src_v.at[0]], rows_l[b], semg[b]).wait()
        pltpu.make_async_copy(xr_hbm.at[dst_v.at[0]], rows_r[b], semg[b]).wait()

    def do_chunk(j, b):
        wait_gather(b)

        @pl.when(j + 1 < n_chunks)
        def _():
            issue_gather(j + 1, b ^ 1)

        accs0 = tuple(jnp.zeros((16,), jnp.float32) for _ in range(K // 16))

        @plsc.parallel_loop(0, C, unroll=4, carry=accs0)
        def accs(ci, acc_in):
            cc = jnp.full((16,), ci, jnp.int32) + iota16
            colv = jnp.where(cc >= C, cc - C, cc)
            attv = plsc.load_gather(att_v, [colv])
            out = []
            for g in range(K // 16):
                a = plsc.load_gather(rows_l[b], [rowid[g], colv])
                r = plsc.load_gather(rows_r[b], [rowid[g], colv])
                t = a + r
                t = jnp.maximum(t, 0.2 * t)
                out.append(acc_in[g] + attv * t)
            return tuple(out)

        exs = [jnp.exp(a) for a in accs]
        col40 = jnp.full((16,), C, jnp.int32)
        for g in range(K // 16):
            plsc.store_scatter(rows_l[b], [rowid[g], col40], exs[g])

        @plsc.parallel_loop(0, C, unroll=4)
        def _scale(ci):
            cc = jnp.full((16,), ci, jnp.int32) + iota16
            colv = jnp.where(cc >= C, cc - C, cc)
            for g in range(K // 16):
                v = plsc.load_gather(rows_l[b], [rowid[g], colv])
                plsc.store_scatter(rows_l[b], [rowid[g], colv], v * exs[g])

        pltpu.sync_copy(rows_l[b], numer_sh.at[dst_v.at[j]], add=True)

    issue_gather(0, 0)

    def outer(i, _):
        for b in range(2):
            do_chunk(2 * i + b, b)
        return 0

    lax.fori_loop(0, n_chunks // 2, outer, 0)
    plsc.subcore_barrier()

    row0 = sid * ROWS_PER_TILE
    pltpu.sync_copy(numer_sh.at[pl.ds(row0, ROWS_PER_TILE), :],
                    numer_out.at[cid, pl.ds(row0, ROWS_PER_TILE), :])


def kernel(x, edge_index, W_l, b_l, W_r, b_r, att, bias_gat, W_lin, b_lin):
    e_tot = edge_index.shape[1] + N
    n_chunks = -(-e_tot // (NWORK * K))
    n_chunks += n_chunks % 2
    e_pad = NWORK * K * n_chunks

    src = edge_index[0]
    dst = edge_index[1]
    loop = jnp.arange(N, dtype=jnp.int32)
    fill = jnp.full((e_pad - e_tot,), NP - 1, jnp.int32)
    src_pad = jnp.concatenate([src, loop, fill]).reshape(NWORK, n_chunks, K)
    dst_pad = jnp.concatenate([dst, loop, fill]).reshape(NWORK, n_chunks, K)
    x_pad = jnp.pad(x, ((0, NP - N), (0, 0)))
    wl_p = jnp.pad(W_l, ((0, 0), (0, CP - C)))
    wr_p = jnp.pad(W_r, ((0, 0), (0, CP - C)))
    wlin_p = jnp.pad(W_lin, ((0, 0), (0, CP - C)))
    bl_p = jnp.pad(b_l, (0, CP - C)).reshape(1, CP)
    br_p = jnp.pad(b_r, (0, CP - C)).reshape(1, CP)
    blin_p = jnp.pad(b_lin, (0, CP - C)).reshape(1, CP)
    att_p = jnp.pad(att, (0, 128 - C))
    bias_p = jnp.pad(bias_gat, (0, CP - C)).reshape(1, CP)

    blk = ROWS_PER_TILE
    grid = NP // blk
    xl_pad, xr_pad, lm_pad = pl.pallas_call(
        _mm_body,
        grid=(grid,),
        in_specs=[
            pl.BlockSpec((blk, F), lambda i: (i, 0)),
            pl.BlockSpec((F, CP), lambda i: (0, 0)),
            pl.BlockSpec((1, CP), lambda i: (0, 0)),
            pl.BlockSpec((F, CP), lambda i: (0, 0)),
            pl.BlockSpec((1, CP), lambda i: (0, 0)),
            pl.BlockSpec((F, CP), lambda i: (0, 0)),
            pl.BlockSpec((1, CP), lambda i: (0, 0)),
        ],
        out_specs=[
            pl.BlockSpec((blk, CP), lambda i: (i, 0)),
            pl.BlockSpec((blk, CP), lambda i: (i, 0)),
            pl.BlockSpec((blk, CP), lambda i: (i, 0)),
        ],
        out_shape=[
            jax.ShapeDtypeStruct((NP, CP), jnp.float32),
            jax.ShapeDtypeStruct((NP, CP), jnp.float32),
            jax.ShapeDtypeStruct((NP, CP), jnp.float32),
        ],
    )(x_pad, wl_p, bl_p, wr_p, br_p, wlin_p, blin_p)

    mesh = plsc.VectorSubcoreMesh(core_axis_name="c", subcore_axis_name="s")
    numer_part = pl.kernel(
        functools.partial(_sc_body, n_chunks),
        out_type=jax.ShapeDtypeStruct((2, NP, CP), jnp.float32),
        mesh=mesh,
        scratch_types=[
            pltpu.VMEM((128,), jnp.float32),
            pltpu.VMEM((n_chunks, K), jnp.int32),
            pltpu.VMEM((n_chunks, K), jnp.int32),
            pltpu.VMEM((K, CP), jnp.float32),
            pltpu.VMEM((K, CP), jnp.float32),
            pltpu.VMEM((K, CP), jnp.float32),
            pltpu.VMEM((K, CP), jnp.float32),
            pltpu.VMEM((K, CP), jnp.float32),
            pltpu.VMEM_SHARED((NP, CP), jnp.float32),
            pltpu.SemaphoreType.DMA,
            pltpu.SemaphoreType.DMA,
        ],
        compiler_params=pltpu.CompilerParams(needs_layout_passes=False,
                                             use_tc_tiling_on_sc=False),
    )(src_pad, dst_pad, xl_pad, xr_pad, att_p)

    out_comb = pl.pallas_call(
        _combine_body,
        grid=(grid,),
        in_specs=[
            pl.BlockSpec((blk, CP), lambda i: (i, 0)),
            pl.BlockSpec((blk, CP), lambda i: (i, 0)),
            pl.BlockSpec((blk, 1), lambda i: (i, 0)),
            pl.BlockSpec((blk, 1), lambda i: (i, 0)),
            pl.BlockSpec((1, CP), lambda i: (0, 0)),
        ],
        out_specs=pl.BlockSpec((blk, CP), lambda i: (i, 0)),
        out_shape=jax.ShapeDtypeStruct((NP, CP), jnp.float32),
    )(numer_part[0], numer_part[1],
      numer_part[0, :, C].reshape(NP, 1), numer_part[1, :, C].reshape(NP, 1),
      bias_p)

    out_gnn = out_comb[:N, :C]
    out_lm = lm_pad[:N, :C]
    return (out_gnn, out_lm, x)


# speedup vs baseline: 20.7154x; 1.0164x over previous
import functools

import jax
import jax.numpy as jnp
from jax import lax
from jax.experimental import pallas as pl
from jax.experimental.pallas import tpu as pltpu
from jax.experimental.pallas import tpu_sc as plsc

N = 10000
F = 128
C = 40
NP = 10240
CP = 48
K = 128
NWORK = 32
ROWS_PER_TILE = NP // 16


def _mm_body(x_ref, wl_ref, bl_ref, wr_ref, br_ref, wlin_ref, blin_ref,
             xl_ref, xr_ref, lm_ref):
    i = pl.program_id(0)
    xb = x_ref[...]
    rows = i * xb.shape[0] + lax.broadcasted_iota(jnp.int32, (xb.shape[0], 1), 0)
    mask = (rows < N).astype(jnp.float32)
    xl_ref[...] = (jnp.dot(xb, wl_ref[...],
                           preferred_element_type=jnp.float32) + bl_ref[...]) * mask
    xr_ref[...] = (jnp.dot(xb, wr_ref[...],
                           preferred_element_type=jnp.float32) + br_ref[...]) * mask
    lm_ref[...] = jnp.dot(xb, wlin_ref[...],
                          preferred_element_type=jnp.float32) + blin_ref[...]


def _combine_body(n0_ref, n1_ref, d0_ref, d1_ref, bias_ref, out_ref):
    denom = d0_ref[...] + d1_ref[...] + 1e-16
    out_ref[...] = (n0_ref[...] + n1_ref[...]) / denom + bias_ref[...]


def _sc_body(n_chunks, src_hbm, dst_hbm, xl_hbm, xr_hbm, att_hbm,
             numer_out,
             att_v, att_rot, src_v, dst_v, rows_l0, rows_l1, rows_r0, rows_r1,
             zbuf, numer_sh, semg0, semg1):
    cid = lax.axis_index("c")
    sid = lax.axis_index("s")
    wid = sid * 2 + cid
    rows_l = (rows_l0, rows_l1)
    rows_r = (rows_r0, rows_r1)
    semg = (semg0, semg1)

    pltpu.sync_copy(att_hbm, att_v)
    pltpu.sync_copy(src_hbm.at[wid], src_v)
    pltpu.sync_copy(dst_hbm.at[wid], dst_v)

    zero16 = jnp.zeros((16,), jnp.float32)

    def zrow(j, _):
        for t in range(CP // 16):
            zbuf[j, pl.ds(t * 16, 16)] = zero16
        return 0

    lax.fori_loop(0, K, zrow, 0)
    for k in range(ROWS_PER_TILE // K):
        pltpu.sync_copy(zbuf, numer_sh.at[pl.ds(sid * ROWS_PER_TILE + k * K, K), :])
    plsc.subcore_barrier()

    iota16 = lax.iota(jnp.int32, 16)
    rowid = [iota16 + 16 * g for g in range(K // 16)]

    def arot(ci, _):
        cc = jnp.full((16,), ci, jnp.int32) + iota16
        colv = jnp.where(cc >= C, cc - C, cc)
        att_rot[pl.ds(ci * 16, 16)] = plsc.load_gather(att_v, [colv])
        return 0

    lax.fori_loop(0, C, arot, 0)

    def issue_gather(j, b):
        pltpu.async_copy(xl_hbm.at[src_v.at[j]], rows_l[b], semg[b])
        pltpu.async_copy(xr_hbm.at[dst_v.at[j]], rows_r[b], semg[b])

    def wait_gather(b):
        pltpu.make_async_copy(xl_hbm.at[src_v.at[0]], rows_l[b], semg[b]).wait()
        pltpu.make_async_copy(xr_hbm.at[dst_v.at[0]], rows_r[b], semg[b]).wait()

    def do_chunk(j, b):
        wait_gather(b)

        @pl.when(j + 1 < n_chunks)
        def _():
            issue_gather(j + 1, b ^ 1)

        accs0 = tuple(jnp.zeros((16,), jnp.float32) for _ in range(K // 16))

        @plsc.parallel_loop(0, C, unroll=8, carry=accs0)
        def accs(ci, acc_in):
            cc = jnp.full((16,), ci, jnp.int32) + iota16
            colv = jnp.where(cc >= C, cc - C, cc)
            attv = att_rot[pl.ds(ci * 16, 16)]
            out = []
            for g in range(K // 16):
                a = plsc.load_gather(rows_l[b], [rowid[g], colv])
                r = plsc.load_gather(rows_r[b], [rowid[g], colv])
                t = a + r
                t = jnp.maximum(t, 0.2 * t)
                out.append(acc_in[g] + attv * t)
            return tuple(out)

        exs = [jnp.exp(a) for a in accs]
        col40 = jnp.full((16,), C, jnp.int32)
        for g in range(K // 16):
            plsc.store_scatter(rows_l[b], [rowid[g], col40], exs[g])

        @plsc.parallel_loop(0, C, unroll=8)
        def _scale(ci):
            cc = jnp.full((16,), ci, jnp.int32) + iota16
            colv = jnp.where(cc >= C, cc - C, cc)
            for g in range(K // 16):
                v = plsc.load_gather(rows_l[b], [rowid[g], colv])
                plsc.store_scatter(rows_l[b], [rowid[g], colv], v * exs[g])

        pltpu.sync_copy(rows_l[b], numer_sh.at[dst_v.at[j]], add=True)

    issue_gather(0, 0)

    def outer(i, _):
        for b in range(2):
            do_chunk(2 * i + b, b)
        return 0

    lax.fori_loop(0, n_chunks // 2, outer, 0)
    plsc.subcore_barrier()

    row0 = sid * ROWS_PER_TILE
    pltpu.sync_copy(numer_sh.at[pl.ds(row0, ROWS_PER_TILE), :],
                    numer_out.at[cid, pl.ds(row0, ROWS_PER_TILE), :])


def kernel(x, edge_index, W_l, b_l, W_r, b_r, att, bias_gat, W_lin, b_lin):
    e_tot = edge_index.shape[1] + N
    n_chunks = -(-e_tot // (NWORK * K))
    n_chunks += n_chunks % 2
    e_pad = NWORK * K * n_chunks

    src = edge_index[0]
    dst = edge_index[1]
    loop = jnp.arange(N, dtype=jnp.int32)
    fill = jnp.full((e_pad - e_tot,), NP - 1, jnp.int32)
    src_pad = jnp.concatenate([src, loop, fill]).reshape(NWORK, n_chunks, K)
    dst_pad = jnp.concatenate([dst, loop, fill]).reshape(NWORK, n_chunks, K)
    x_pad = jnp.pad(x, ((0, NP - N), (0, 0)))
    wl_p = jnp.pad(W_l, ((0, 0), (0, CP - C)))
    wr_p = jnp.pad(W_r, ((0, 0), (0, CP - C)))
    wlin_p = jnp.pad(W_lin, ((0, 0), (0, CP - C)))
    bl_p = jnp.pad(b_l, (0, CP - C)).reshape(1, CP)
    br_p = jnp.pad(b_r, (0, CP - C)).reshape(1, CP)
    blin_p = jnp.pad(b_lin, (0, CP - C)).reshape(1, CP)
    att_p = jnp.pad(att, (0, 128 - C))
    bias_p = jnp.pad(bias_gat, (0, CP - C)).reshape(1, CP)

    blk = ROWS_PER_TILE
    grid = NP // blk
    xl_pad, xr_pad, lm_pad = pl.pallas_call(
        _mm_body,
        grid=(grid,),
        in_specs=[
            pl.BlockSpec((blk, F), lambda i: (i, 0)),
            pl.BlockSpec((F, CP), lambda i: (0, 0)),
            pl.BlockSpec((1, CP), lambda i: (0, 0)),
            pl.BlockSpec((F, CP), lambda i: (0, 0)),
            pl.BlockSpec((1, CP), lambda i: (0, 0)),
            pl.BlockSpec((F, CP), lambda i: (0, 0)),
            pl.BlockSpec((1, CP), lambda i: (0, 0)),
        ],
        out_specs=[
            pl.BlockSpec((blk, CP), lambda i: (i, 0)),
            pl.BlockSpec((blk, CP), lambda i: (i, 0)),
            pl.BlockSpec((blk, CP), lambda i: (i, 0)),
        ],
        out_shape=[
            jax.ShapeDtypeStruct((NP, CP), jnp.float32),
            jax.ShapeDtypeStruct((NP, CP), jnp.float32),
            jax.ShapeDtypeStruct((NP, CP), jnp.float32),
        ],
    )(x_pad, wl_p, bl_p, wr_p, br_p, wlin_p, blin_p)

    mesh = plsc.VectorSubcoreMesh(core_axis_name="c", subcore_axis_name="s")
    numer_part = pl.kernel(
        functools.partial(_sc_body, n_chunks),
        out_type=jax.ShapeDtypeStruct((2, NP, CP), jnp.float32),
        mesh=mesh,
        scratch_types=[
            pltpu.VMEM((128,), jnp.float32),
            pltpu.VMEM((16 * C,), jnp.float32),
            pltpu.VMEM((n_chunks, K), jnp.int32),
            pltpu.VMEM((n_chunks, K), jnp.int32),
            pltpu.VMEM((K, CP), jnp.float32),
            pltpu.VMEM((K, CP), jnp.float32),
            pltpu.VMEM((K, CP), jnp.float32),
            pltpu.VMEM((K, CP), jnp.float32),
            pltpu.VMEM((K, CP), jnp.float32),
            pltpu.VMEM_SHARED((NP, CP), jnp.float32),
            pltpu.SemaphoreType.DMA,
            pltpu.SemaphoreType.DMA,
        ],
        compiler_params=pltpu.CompilerParams(needs_layout_passes=False,
                                             use_tc_tiling_on_sc=False),
    )(src_pad, dst_pad, xl_pad, xr_pad, att_p)

    out_comb = pl.pallas_call(
        _combine_body,
        grid=(grid,),
        in_specs=[
            pl.BlockSpec((blk, CP), lambda i: (i, 0)),
            pl.BlockSpec((blk, CP), lambda i: (i, 0)),
            pl.BlockSpec((blk, 1), lambda i: (i, 0)),
            pl.BlockSpec((blk, 1), lambda i: (i, 0)),
            pl.BlockSpec((1, CP), lambda i: (0, 0)),
        ],
        out_specs=pl.BlockSpec((blk, CP), lambda i: (i, 0)),
        out_shape=jax.ShapeDtypeStruct((NP, CP), jnp.float32),
    )(numer_part[0], numer_part[1],
      numer_part[0, :, C].reshape(NP, 1), numer_part[1, :, C].reshape(NP, 1),
      bias_p)

    out_gnn = out_comb[:N, :C]
    out_lm = lm_pad[:N, :C]
    return (out_gnn, out_lm, x)
